# Initial kernel scaffold; baseline (speedup 1.0000x reference)
import functools

import jax
import jax.numpy as jnp
from jax import lax
from jax.experimental import pallas as pl
from jax.experimental.pallas import tpu as pltpu
from jax.experimental.pallas import tpu_sc as plsc

NC = 2
NS = 16
NW = NC * NS

CHUNK = 1024


@functools.lru_cache(maxsize=None)
def _make_gather(n_flat: int, dim: int):
    assert n_flat % NW == 0
    per_w = n_flat // NW
    assert per_w % CHUNK == 0
    n_chunks = per_w // CHUNK

    mesh = plsc.VectorSubcoreMesh(core_axis_name="c", subcore_axis_name="s")

    @functools.partial(
        pl.kernel,
        mesh=mesh,
        out_type=jax.ShapeDtypeStruct((n_flat, dim), jnp.float32),
        scratch_types=[
            pltpu.VMEM((per_w,), jnp.int32),
            pltpu.VMEM((CHUNK, dim), jnp.float32),
            pltpu.SemaphoreType.DMA,
        ],
    )
    def gather_kernel(idx_hbm, table_hbm, out_hbm, idx_v, rows_v, sem):
        wid = lax.axis_index("s") * NC + lax.axis_index("c")
        base = wid * per_w
        pltpu.sync_copy(idx_hbm.at[pl.ds(base, per_w)], idx_v)

        def body(c, carry):
            off = c * CHUNK
            idx_chunk = idx_v.at[pl.ds(off, CHUNK)]
            pltpu.async_copy(table_hbm.at[idx_chunk], rows_v, sem).wait()
            pltpu.sync_copy(rows_v, out_hbm.at[pl.ds(base + off, CHUNK)])
            return carry

        lax.fori_loop(0, n_chunks, body, 0)

    return gather_kernel


def kernel(x, table):
    b, t = x.shape
    flat = jnp.reshape(x, (-1,)).astype(jnp.int32)
    out = _make_gather(b * t, table.shape[1])(flat, table)
    return jnp.reshape(out, (b, t, table.shape[1]))


# speedup vs baseline: 1.4792x; 1.4792x over previous
import functools

import jax
import jax.numpy as jnp
from jax import lax
from jax.experimental import pallas as pl
from jax.experimental.pallas import tpu as pltpu
from jax.experimental.pallas import tpu_sc as plsc

NC = 2
NS = 16
NW = NC * NS

CHUNK = 1024


@functools.lru_cache(maxsize=None)
def _make_gather(n_flat: int, dim: int):
    assert n_flat % NW == 0
    per_w = n_flat // NW
    assert per_w % CHUNK == 0
    n_chunks = per_w // CHUNK

    mesh = plsc.VectorSubcoreMesh(core_axis_name="c", subcore_axis_name="s")

    @functools.partial(
        pl.kernel,
        mesh=mesh,
        compiler_params=pltpu.CompilerParams(use_tc_tiling_on_sc=False),
        out_type=jax.ShapeDtypeStruct((n_flat, dim), jnp.float32),
        scratch_types=[
            pltpu.VMEM((per_w,), jnp.int32),
            pltpu.VMEM((CHUNK, dim), jnp.float32),
            pltpu.SemaphoreType.DMA,
        ],
    )
    def gather_kernel(idx_hbm, table_hbm, out_hbm, idx_v, rows_v, sem):
        wid = lax.axis_index("s") * NC + lax.axis_index("c")
        base = wid * per_w
        pltpu.sync_copy(idx_hbm.at[pl.ds(base, per_w)], idx_v)

        def body(c, carry):
            off = c * CHUNK
            idx_chunk = idx_v.at[pl.ds(off, CHUNK)]
            pltpu.async_copy(table_hbm.at[idx_chunk], rows_v, sem).wait()
            pltpu.sync_copy(rows_v, out_hbm.at[pl.ds(base + off, CHUNK)])
            return carry

        lax.fori_loop(0, n_chunks, body, 0)

    return gather_kernel


def kernel(x, table):
    b, t = x.shape
    flat = jnp.reshape(x, (-1,)).astype(jnp.int32)
    out = _make_gather(b * t, table.shape[1])(flat, table)
    return jnp.reshape(out, (b, t, table.shape[1]))


# speedup vs baseline: 1.4971x; 1.0121x over previous
import functools

import jax
import jax.numpy as jnp
from jax import lax
from jax.experimental import pallas as pl
from jax.experimental.pallas import tpu as pltpu
from jax.experimental.pallas import tpu_sc as plsc

NC = 2
NS = 16
NW = NC * NS

CHUNK = 512
NBUF = 5


@functools.lru_cache(maxsize=None)
def _make_gather(n_flat: int, dim: int):
    assert n_flat % NW == 0
    per_w = n_flat // NW
    assert per_w % (CHUNK * NBUF) == 0
    n_rounds = per_w // (CHUNK * NBUF)

    mesh = plsc.VectorSubcoreMesh(core_axis_name="c", subcore_axis_name="s")

    @functools.partial(
        pl.kernel,
        mesh=mesh,
        compiler_params=pltpu.CompilerParams(use_tc_tiling_on_sc=False),
        out_type=jax.ShapeDtypeStruct((n_flat, dim), jnp.float32),
        scratch_types=[
            pltpu.VMEM((per_w,), jnp.int32),
            *[pltpu.VMEM((CHUNK, dim), jnp.float32) for _ in range(NBUF)],
            *[pltpu.SemaphoreType.DMA for _ in range(NBUF)],
            *[pltpu.SemaphoreType.DMA for _ in range(NBUF)],
        ],
    )
    def gather_kernel(idx_hbm, table_hbm, out_hbm, idx_v, *bufs_and_sems):
        rows = bufs_and_sems[:NBUF]
        sem_g = bufs_and_sems[NBUF:2 * NBUF]
        sem_w = bufs_and_sems[2 * NBUF:3 * NBUF]

        wid = lax.axis_index("s") * NC + lax.axis_index("c")
        base = wid * per_w
        pltpu.sync_copy(idx_hbm.at[pl.ds(base, per_w)], idx_v)

        def body(i, carry):
            rbase = i * (CHUNK * NBUF)
            for b in range(NBUF):
                off = rbase + b * CHUNK

                @pl.when(i > 0)
                def _wait_prev_write():
                    pltpu.make_async_copy(
                        rows[b], out_hbm.at[pl.ds(base + off, CHUNK)], sem_w[b]
                    ).wait()

                pltpu.async_copy(
                    table_hbm.at[idx_v.at[pl.ds(off, CHUNK)]], rows[b], sem_g[b]
                )
            for b in range(NBUF):
                off = rbase + b * CHUNK
                pltpu.make_async_copy(
                    table_hbm.at[idx_v.at[pl.ds(off, CHUNK)]], rows[b], sem_g[b]
                ).wait()
                pltpu.async_copy(
                    rows[b], out_hbm.at[pl.ds(base + off, CHUNK)], sem_w[b]
                )
            return carry

        lax.fori_loop(0, n_rounds, body, 0)
        for b in range(NBUF):
            pltpu.make_async_copy(
                rows[b], out_hbm.at[pl.ds(base, CHUNK)], sem_w[b]
            ).wait()

    return gather_kernel


def kernel(x, table):
    b, t = x.shape
    flat = jnp.reshape(x, (-1,)).astype(jnp.int32)
    out = _make_gather(b * t, table.shape[1])(flat, table)
    return jnp.reshape(out, (b, t, table.shape[1]))


# speedup vs baseline: 2.0441x; 1.3654x over previous
import functools

import jax
import jax.numpy as jnp
from jax import lax
from jax.experimental import pallas as pl
from jax.experimental.pallas import tpu as pltpu
from jax.experimental.pallas import tpu_sc as plsc

NC = 2
NS = 16
NW = NC * NS

CHUNK = 512
NBUF = 5


@functools.lru_cache(maxsize=None)
def _make_gather(n_flat: int, dim: int):
    assert n_flat % NW == 0
    per_w = n_flat // NW
    assert per_w % (CHUNK * NBUF) == 0
    n_rounds = per_w // (CHUNK * NBUF)

    mesh = plsc.VectorSubcoreMesh(core_axis_name="c", subcore_axis_name="s")

    @functools.partial(
        pl.kernel,
        mesh=mesh,
        compiler_params=pltpu.CompilerParams(use_tc_tiling_on_sc=False),
        out_type=jax.ShapeDtypeStruct((n_flat, 128), jnp.float32),
        scratch_types=[
            pltpu.VMEM((per_w,), jnp.int32),
            *[pltpu.VMEM((CHUNK, dim), jnp.float32) for _ in range(NBUF)],
            *[pltpu.SemaphoreType.DMA for _ in range(NBUF)],
            *[pltpu.SemaphoreType.DMA for _ in range(NBUF)],
        ],
    )
    def gather_kernel(idx_hbm, table_hbm, out_hbm, idx_v, *bufs_and_sems):
        rows = bufs_and_sems[:NBUF]
        sem_g = bufs_and_sems[NBUF:2 * NBUF]
        sem_w = bufs_and_sems[2 * NBUF:3 * NBUF]

        wid = lax.axis_index("s") * NC + lax.axis_index("c")
        base = wid * per_w
        pltpu.sync_copy(idx_hbm.at[pl.ds(base, per_w)], idx_v)

        def body(i, carry):
            rbase = i * (CHUNK * NBUF)
            for b in range(NBUF):
                off = rbase + b * CHUNK

                @pl.when(i > 0)
                def _wait_prev_write():
                    pltpu.make_async_copy(
                        rows[b],
                        out_hbm.at[pl.ds(base + off, CHUNK), pl.ds(0, dim)],
                        sem_w[b],
                    ).wait()

                pltpu.async_copy(
                    table_hbm.at[idx_v.at[pl.ds(off, CHUNK)]], rows[b], sem_g[b]
                )
            for b in range(NBUF):
                off = rbase + b * CHUNK
                pltpu.make_async_copy(
                    table_hbm.at[idx_v.at[pl.ds(off, CHUNK)]], rows[b], sem_g[b]
                ).wait()
                pltpu.async_copy(
                    rows[b],
                    out_hbm.at[pl.ds(base + off, CHUNK), pl.ds(0, dim)],
                    sem_w[b],
                )
            return carry

        lax.fori_loop(0, n_rounds, body, 0)
        for b in range(NBUF):
            pltpu.make_async_copy(
                rows[b], out_hbm.at[pl.ds(base, CHUNK), pl.ds(0, dim)], sem_w[b]
            ).wait()

    return gather_kernel


def kernel(x, table):
    b, t = x.shape
    dim = table.shape[1]
    flat = jnp.reshape(x, (-1,)).astype(jnp.int32)
    out128 = _make_gather(b * t, dim)(flat, table)
    return jnp.reshape(out128[:, :dim], (b, t, dim))
